# Initial kernel scaffold; baseline (speedup 1.0000x reference)
import jax
import jax.numpy as jnp
from jax.experimental import pallas as pl


def kernel(hidden_states, router_weight, gate_up_proj, down_proj, shared_gate_w, shared_up_w, shared_down_w):
    raise NotImplementedError("write your pallas kernel here")


# speedup vs baseline: 1.3235x; 1.3235x over previous
import jax
import jax.numpy as jnp
from jax.experimental import pallas as pl
from jax.experimental.pallas import tpu as pltpu

H = 2048
I = 2048
NE = 8
T = 2048

BT = 256
NT = T // BT + NE
TPAD = NT * BT
CJ = 1024
NJ = I // CJ

BTR = 256
BTS = 256
CJS = 512
NJS = I // CJS


def _router_body(x_ref, wr_ref, logits_ref, eidx_ref, xs_ref):
    x = x_ref[...]
    logits = jnp.dot(x, wr_ref[...], preferred_element_type=jnp.float32)
    m = jnp.max(logits, axis=1, keepdims=True)
    a = jnp.argmax(logits, axis=1).astype(jnp.int32)
    logits_ref[...] = logits
    eidx_ref[...] = a[:, None]
    xs_ref[...] = x * jax.nn.sigmoid(m)


def _gmm_a_body(te_ref, x_ref, g_ref, u_ref, h_ref):
    x = x_ref[...]
    g = jnp.dot(x, g_ref[0], preferred_element_type=jnp.float32)
    u = jnp.dot(x, u_ref[0], preferred_element_type=jnp.float32)
    h_ref[...] = u * g * jax.nn.sigmoid(g)


def _gmm_b_body(te_ref, h_ref, d_ref, o_ref):
    o_ref[...] = jnp.dot(h_ref[...], d_ref[0], preferred_element_type=jnp.float32)


def _shared_body(x_ref, gw_ref, uw_ref, dw_ref, r_ref, o_ref):
    j = pl.program_id(1)
    x = x_ref[...]
    g = jnp.dot(x, gw_ref[...], preferred_element_type=jnp.float32)
    u = jnp.dot(x, uw_ref[...], preferred_element_type=jnp.float32)
    s1 = u * g * jax.nn.sigmoid(g)
    part = jnp.dot(s1, dw_ref[...], preferred_element_type=jnp.float32)

    @pl.when(j == 0)
    def _():
        o_ref[...] = r_ref[...] + part

    @pl.when(j != 0)
    def _():
        o_ref[...] += part


def kernel(hidden_states, router_weight, gate_up_proj, down_proj,
           shared_gate_w, shared_up_w, shared_down_w):
    x = hidden_states.reshape(T, H)

    logits, eidx2, xs = pl.pallas_call(
        _router_body,
        grid=(T // BTR,),
        in_specs=[pl.BlockSpec((BTR, H), lambda i: (i, 0)),
                  pl.BlockSpec((H, NE), lambda i: (0, 0))],
        out_specs=[pl.BlockSpec((BTR, NE), lambda i: (i, 0)),
                   pl.BlockSpec((BTR, 1), lambda i: (i, 0)),
                   pl.BlockSpec((BTR, H), lambda i: (i, 0))],
        out_shape=[jax.ShapeDtypeStruct((T, NE), jnp.float32),
                   jax.ShapeDtypeStruct((T, 1), jnp.int32),
                   jax.ShapeDtypeStruct((T, H), jnp.float32)],
    )(x, router_weight)
    eidx = eidx2[:, 0]

    oh = eidx[:, None] == jnp.arange(NE, dtype=jnp.int32)[None, :]
    counts = jnp.sum(oh.astype(jnp.int32), axis=0)
    padded = ((counts + BT - 1) // BT) * BT
    ends = jnp.cumsum(padded)
    starts = ends - padded
    rank = jnp.take_along_axis(jnp.cumsum(oh.astype(jnp.int32), axis=0),
                               eidx[:, None], axis=1)[:, 0] - 1
    pos = starts[eidx] + rank
    perm = jnp.zeros((TPAD,), jnp.int32).at[pos].set(
        jnp.arange(T, dtype=jnp.int32))
    tile_base = jnp.arange(NT, dtype=jnp.int32) * BT
    te = jnp.minimum(jnp.searchsorted(ends, tile_base, side='right'),
                     NE - 1).astype(jnp.int32)

    x_sorted = jnp.take(xs, perm, axis=0)

    grid_a = pltpu.PrefetchScalarGridSpec(
        num_scalar_prefetch=1,
        grid=(NJ, NT),
        in_specs=[pl.BlockSpec((BT, H), lambda j, i, te: (i, 0)),
                  pl.BlockSpec((1, H, CJ), lambda j, i, te: (te[i], 0, j)),
                  pl.BlockSpec((1, H, CJ), lambda j, i, te: (te[i], 0, NJ + j))],
        out_specs=pl.BlockSpec((BT, CJ), lambda j, i, te: (i, j)),
    )
    h = pl.pallas_call(
        _gmm_a_body, grid_spec=grid_a,
        out_shape=jax.ShapeDtypeStruct((TPAD, I), jnp.float32),
    )(te, x_sorted, gate_up_proj, gate_up_proj)

    grid_b = pltpu.PrefetchScalarGridSpec(
        num_scalar_prefetch=1,
        grid=(NT,),
        in_specs=[pl.BlockSpec((BT, I), lambda i, te: (i, 0)),
                  pl.BlockSpec((1, I, H), lambda i, te: (te[i], 0, 0))],
        out_specs=pl.BlockSpec((BT, H), lambda i, te: (i, 0)),
    )
    routed_sorted = pl.pallas_call(
        _gmm_b_body, grid_spec=grid_b,
        out_shape=jax.ShapeDtypeStruct((TPAD, H), jnp.float32),
    )(te, h, down_proj)

    routed_tok = jnp.take(routed_sorted, pos, axis=0)

    out = pl.pallas_call(
        _shared_body,
        grid=(T // BTS, NJS),
        in_specs=[pl.BlockSpec((BTS, H), lambda i, j: (i, 0)),
                  pl.BlockSpec((H, CJS), lambda i, j: (0, j)),
                  pl.BlockSpec((H, CJS), lambda i, j: (0, j)),
                  pl.BlockSpec((CJS, H), lambda i, j: (j, 0)),
                  pl.BlockSpec((BTS, H), lambda i, j: (i, 0))],
        out_specs=pl.BlockSpec((BTS, H), lambda i, j: (i, 0)),
        out_shape=jax.ShapeDtypeStruct((T, H), jnp.float32),
    )(x, shared_gate_w, shared_up_w, shared_down_w, routed_tok)

    return out, logits


# speedup vs baseline: 1.5124x; 1.1428x over previous
import jax
import jax.numpy as jnp
from jax.experimental import pallas as pl
from jax.experimental.pallas import tpu as pltpu

H = 2048
I = 2048
NE = 8
T = 2048

BT = 256
NT = T // BT + NE
TPAD = NT * BT
CJ = 1024
NJ = I // CJ

BTR = 256
BTS = 256
CJS = 1024
NJS = I // CJS


def _router_body(x_ref, wr_ref, logits_ref, eidx_ref, xs_ref):
    x = x_ref[...]
    logits = jnp.dot(x, wr_ref[...], preferred_element_type=jnp.float32)
    m = jnp.max(logits, axis=1, keepdims=True)
    a = jnp.argmax(logits, axis=1).astype(jnp.int32)
    logits_ref[...] = logits
    eidx_ref[...] = a[:, None]
    xs_ref[...] = x * jax.nn.sigmoid(m)


def _gmm_a_body(te_ref, x_ref, g_ref, u_ref, h_ref):
    x = x_ref[...]
    g = jnp.dot(x, g_ref[0], preferred_element_type=jnp.float32)
    u = jnp.dot(x, u_ref[0], preferred_element_type=jnp.float32)
    h_ref[...] = u * g * jax.nn.sigmoid(g)


def _gmm_b_body(te_ref, h_ref, d_ref, o_ref):
    o_ref[...] = jnp.dot(h_ref[...], d_ref[0], preferred_element_type=jnp.float32)


def _shared_a_body(x_ref, gw_ref, uw_ref, s1_ref):
    x = x_ref[...]
    g = jnp.dot(x, gw_ref[...], preferred_element_type=jnp.float32)
    u = jnp.dot(x, uw_ref[...], preferred_element_type=jnp.float32)
    s1_ref[...] = u * g * jax.nn.sigmoid(g)


def _shared_b_body(s1_ref, dw_ref, r_ref, o_ref):
    o_ref[...] = r_ref[...] + jnp.dot(
        s1_ref[...], dw_ref[...], preferred_element_type=jnp.float32)


def kernel(hidden_states, router_weight, gate_up_proj, down_proj,
           shared_gate_w, shared_up_w, shared_down_w):
    x = hidden_states.reshape(T, H)

    logits, eidx2, xs = pl.pallas_call(
        _router_body,
        grid=(T // BTR,),
        in_specs=[pl.BlockSpec((BTR, H), lambda i: (i, 0)),
                  pl.BlockSpec((H, NE), lambda i: (0, 0))],
        out_specs=[pl.BlockSpec((BTR, NE), lambda i: (i, 0)),
                   pl.BlockSpec((BTR, 1), lambda i: (i, 0)),
                   pl.BlockSpec((BTR, H), lambda i: (i, 0))],
        out_shape=[jax.ShapeDtypeStruct((T, NE), jnp.float32),
                   jax.ShapeDtypeStruct((T, 1), jnp.int32),
                   jax.ShapeDtypeStruct((T, H), jnp.float32)],
    )(x, router_weight)
    eidx = eidx2[:, 0]

    oh = eidx[:, None] == jnp.arange(NE, dtype=jnp.int32)[None, :]
    counts = jnp.sum(oh.astype(jnp.int32), axis=0)
    padded = ((counts + BT - 1) // BT) * BT
    ends = jnp.cumsum(padded)
    starts = ends - padded
    rank = jnp.take_along_axis(jnp.cumsum(oh.astype(jnp.int32), axis=0),
                               eidx[:, None], axis=1)[:, 0] - 1
    pos = starts[eidx] + rank
    perm = jnp.zeros((TPAD,), jnp.int32).at[pos].set(
        jnp.arange(T, dtype=jnp.int32))
    tile_base = jnp.arange(NT, dtype=jnp.int32) * BT
    te = jnp.minimum(jnp.searchsorted(ends, tile_base, side='right'),
                     NE - 1).astype(jnp.int32)

    x_sorted = jnp.take(xs, perm, axis=0)

    grid_a = pltpu.PrefetchScalarGridSpec(
        num_scalar_prefetch=1,
        grid=(NJ, NT),
        in_specs=[pl.BlockSpec((BT, H), lambda j, i, te: (i, 0)),
                  pl.BlockSpec((1, H, CJ), lambda j, i, te: (te[i], 0, j)),
                  pl.BlockSpec((1, H, CJ), lambda j, i, te: (te[i], 0, NJ + j))],
        out_specs=pl.BlockSpec((BT, CJ), lambda j, i, te: (i, j)),
    )
    h = pl.pallas_call(
        _gmm_a_body, grid_spec=grid_a,
        out_shape=jax.ShapeDtypeStruct((TPAD, I), jnp.float32),
    )(te, x_sorted, gate_up_proj, gate_up_proj)

    grid_b = pltpu.PrefetchScalarGridSpec(
        num_scalar_prefetch=1,
        grid=(NT,),
        in_specs=[pl.BlockSpec((BT, I), lambda i, te: (i, 0)),
                  pl.BlockSpec((1, I, H), lambda i, te: (te[i], 0, 0))],
        out_specs=pl.BlockSpec((BT, H), lambda i, te: (i, 0)),
    )
    routed_sorted = pl.pallas_call(
        _gmm_b_body, grid_spec=grid_b,
        out_shape=jax.ShapeDtypeStruct((TPAD, H), jnp.float32),
    )(te, h, down_proj)

    routed_tok = jnp.take(routed_sorted, pos, axis=0)

    s1 = pl.pallas_call(
        _shared_a_body,
        grid=(NJS, T // BTS),
        in_specs=[pl.BlockSpec((BTS, H), lambda j, i: (i, 0)),
                  pl.BlockSpec((H, CJS), lambda j, i: (0, j)),
                  pl.BlockSpec((H, CJS), lambda j, i: (0, j))],
        out_specs=pl.BlockSpec((BTS, CJS), lambda j, i: (i, j)),
        out_shape=jax.ShapeDtypeStruct((T, I), jnp.float32),
    )(x, shared_gate_w, shared_up_w)

    out = pl.pallas_call(
        _shared_b_body,
        grid=(T // BTS,),
        in_specs=[pl.BlockSpec((BTS, I), lambda i: (i, 0)),
                  pl.BlockSpec((I, H), lambda i: (0, 0)),
                  pl.BlockSpec((BTS, H), lambda i: (i, 0))],
        out_specs=pl.BlockSpec((BTS, H), lambda i: (i, 0)),
        out_shape=jax.ShapeDtypeStruct((T, H), jnp.float32),
    )(s1, shared_down_w, routed_tok)

    return out, logits
